# baseline (speedup 1.0000x reference)
import functools

import jax
import jax.numpy as jnp
from jax import lax
from jax.experimental import pallas as pl
from jax.experimental.pallas import tpu as pltpu
from jax.experimental.pallas import tpu_sc as plsc

_N = 10000
_E = 320000
_D = 128
_R = 8
_NR = _N * _R

_NC = 2
_NS = 16
_NW = _NC * _NS
_EPW = _E // _NW
_CH16 = _EPW // 16
_SEG = _NR // _NS
_ZB = ((_SEG + 15) // 16) * 16
_NCHUNK = 5
_CHW = _EPW // _NCHUNK

_BN = 2000
_GRID = _N // _BN

_mesh = plsc.VectorSubcoreMesh(core_axis_name="c", subcore_axis_name="s")


@functools.partial(
    pl.kernel,
    out_type=[
        jax.ShapeDtypeStruct((_NC * _NR,), jnp.float32),
        jax.ShapeDtypeStruct((_E,), jnp.int32),
        jax.ShapeDtypeStruct((_E,), jnp.int32),
    ],
    mesh=_mesh,
    scratch_types=[
        pltpu.VMEM((_EPW,), jnp.int32),
        pltpu.VMEM((_EPW,), jnp.int32),
        pltpu.VMEM((_EPW,), jnp.int32),
        pltpu.VMEM((_EPW,), jnp.float32),
        pltpu.VMEM((_ZB,), jnp.float32),
        pltpu.VMEM_SHARED((_NR,), jnp.float32),
        pltpu.SemaphoreType.DMA,
        pltpu.SemaphoreType.DMA,
    ],
)
def _sc_counts(src_hbm, dst_hbm, rel_hbm,
               c_out, gi_out, si_out,
               gidx, sidx, rel_v, ones_v, zbuf, c_sh, sem, sem2):
    cid = lax.axis_index("c")
    sid = lax.axis_index("s")
    wid = sid * _NC + cid

    base = wid * _EPW
    cp_src = pltpu.async_copy(src_hbm.at[pl.ds(base, _EPW)], gidx, sem2)
    cp_dst = pltpu.async_copy(dst_hbm.at[pl.ds(base, _EPW)], sidx, sem2)
    cp_rel = pltpu.async_copy(rel_hbm.at[pl.ds(base, _EPW)], rel_v, sem2)

    def _fill_zero(i, carry):
        zbuf[pl.ds(i * 16, 16)] = jnp.zeros((16,), jnp.float32)
        return carry

    lax.fori_loop(0, _ZB // 16, _fill_zero, 0, unroll=4)

    def _fill_one(i, carry):
        ones_v[pl.ds(i * 16, 16)] = jnp.full((16,), 1.0, jnp.float32)
        return carry

    lax.fori_loop(0, _CH16, _fill_one, 0, unroll=4)

    off = sid * _SEG
    pltpu.sync_copy(zbuf.at[pl.ds(0, _SEG)], c_sh.at[pl.ds(off, _SEG)])

    cp_src.wait()
    cp_dst.wait()
    cp_rel.wait()

    def _mk_sidx(i, carry):
        sl = pl.ds(i * 16, 16)
        sidx[sl] = sidx[sl] * _R + rel_v[sl]
        return carry

    lax.fori_loop(0, _CH16, _mk_sidx, 0, unroll=4)
    cp_si = pltpu.async_copy(sidx, si_out.at[pl.ds(base, _EPW)], sem2)

    plsc.subcore_barrier()

    sct = pltpu.async_copy(ones_v, c_sh.at[sidx], sem, add=True)

    def _mk_gidx(i, carry):
        sl = pl.ds(i * 16, 16)
        gidx[sl] = gidx[sl] * _D + rel_v[sl]
        return carry

    lax.fori_loop(0, _CH16, _mk_gidx, 0, unroll=4)
    cp_gi = pltpu.async_copy(gidx, gi_out.at[pl.ds(base, _EPW)], sem2)

    sct.wait()
    cp_gi.wait()
    cp_si.wait()
    plsc.subcore_barrier()

    oo = cid * _NR + off
    pltpu.sync_copy(c_sh.at[pl.ds(off, _SEG)], ones_v.at[pl.ds(0, _SEG)])
    pltpu.sync_copy(ones_v.at[pl.ds(0, _SEG)], c_out.at[pl.ds(oo, _SEG)])


@functools.partial(
    pl.kernel,
    out_type=jax.ShapeDtypeStruct((_NC * _NR,), jnp.float32),
    mesh=_mesh,
    scratch_types=[
        pltpu.VMEM((_NCHUNK, 1, _CHW), jnp.int32),
        pltpu.VMEM((_NCHUNK, 1, _CHW), jnp.int32),
        pltpu.VMEM((_NCHUNK, 1, _CHW), jnp.float32),
        pltpu.VMEM((_ZB,), jnp.float32),
        pltpu.VMEM_SHARED((_NR,), jnp.float32),
        pltpu.SemaphoreType.DMA,
        pltpu.SemaphoreType.DMA,
    ],
)
def _sc_sums(gi_hbm, si_hbm, z_hbm, s_out,
             gidx, sidx, vals, zbuf, s_sh, sem, sem2):
    cid = lax.axis_index("c")
    sid = lax.axis_index("s")
    wid = sid * _NC + cid

    base = wid * _EPW
    cps = []
    for k in range(_NCHUNK):
        cps.append(pltpu.async_copy(
            gi_hbm.at[pl.ds(base + k * _CHW, _CHW)], gidx.at[k, 0], sem2))
        cps.append(pltpu.async_copy(
            si_hbm.at[pl.ds(base + k * _CHW, _CHW)], sidx.at[k, 0], sem2))

    def _fill_zero(i, carry):
        zbuf[pl.ds(i * 16, 16)] = jnp.zeros((16,), jnp.float32)
        return carry

    lax.fori_loop(0, _ZB // 16, _fill_zero, 0, unroll=4)

    off = sid * _SEG
    pltpu.sync_copy(zbuf.at[pl.ds(0, _SEG)], s_sh.at[pl.ds(off, _SEG)])

    for cp in cps:
        cp.wait()

    gats = [pltpu.async_copy(z_hbm.at[gidx.at[k, 0]], vals.at[k, 0], sem)
            for k in range(2)]

    plsc.subcore_barrier()

    scts = []
    for k in range(_NCHUNK):
        if k + 2 < _NCHUNK:
            gats.append(pltpu.async_copy(
                z_hbm.at[gidx.at[k + 2, 0]], vals.at[k + 2, 0], sem))
        gats[k].wait()
        scts.append(pltpu.async_copy(
            vals.at[k, 0], s_sh.at[sidx.at[k, 0]], sem2, add=True))
    for s in scts:
        s.wait()

    plsc.subcore_barrier()

    oo = cid * _NR + off
    pltpu.sync_copy(s_sh.at[pl.ds(off, _SEG)], zbuf.at[pl.ds(0, _SEG)])
    pltpu.sync_copy(zbuf.at[pl.ds(0, _SEG)], s_out.at[pl.ds(oo, _SEG)])


_NPT = 320
_NLAST = _N - _NPT
_NV = _NPT * _R


@functools.partial(
    pl.kernel,
    out_type=[
        jax.ShapeDtypeStruct((_N,), jnp.float32),
        jax.ShapeDtypeStruct((_N * _D,), jnp.float32),
    ],
    mesh=_mesh,
    scratch_types=[
        pltpu.VMEM((_NV,), jnp.float32),
        pltpu.VMEM((_NV,), jnp.float32),
        pltpu.VMEM((_NV,), jnp.float32),
        pltpu.VMEM((_NV,), jnp.float32),
        pltpu.VMEM((_NV,), jnp.float32),
        pltpu.VMEM((_NV,), jnp.int32),
        pltpu.VMEM((_NPT,), jnp.float32),
        pltpu.VMEM((_NPT,), jnp.int32),
        pltpu.VMEM((_NPT,), jnp.int32),
        pltpu.VMEM((_NPT,), jnp.float32),
        pltpu.VMEM((16,), jnp.float32),
        pltpu.VMEM_SHARED((_NS * _NPT,), jnp.float32),
        pltpu.SemaphoreType.DMA,
        pltpu.SemaphoreType.DMA,
    ],
)
def _sc_score(s_hbm, c_hbm, z_hbm, b_hbm, sc_out, sp_out,
              sa, sb, ca, cb, rat, ridx, aggv, xidx, pidx, xrv, bv, agg_sh,
              sem, sem2):
    cid = lax.axis_index("c")
    sid = lax.axis_index("s")
    wid = sid * _NC + cid
    n0 = jnp.minimum(wid * _NPT, _NLAST)

    cps = [
        pltpu.async_copy(s_hbm.at[pl.ds(n0 * _R, _NV)], sa, sem2),
        pltpu.async_copy(s_hbm.at[pl.ds(_NR + n0 * _R, _NV)], sb, sem2),
        pltpu.async_copy(c_hbm.at[pl.ds(n0 * _R, _NV)], ca, sem2),
        pltpu.async_copy(c_hbm.at[pl.ds(_NR + n0 * _R, _NV)], cb, sem2),
        pltpu.async_copy(b_hbm, bv, sem2),
    ]

    iota16 = lax.iota(jnp.int32, 16)

    def _mk_xidx(g, carry):
        t = (iota16 + g * 16 + n0) * _D
        xidx[pl.ds(g * 16, 16)] = t + _R
        pidx[pl.ds(g * 16, 16)] = t
        return carry

    lax.fori_loop(0, _NPT // 16, _mk_xidx, 0, unroll=4)
    gat = pltpu.async_copy(z_hbm.at[xidx], xrv, sem)

    rbase = sid * _NPT

    def _mk_ridx(c, carry):
        ii = iota16 + c * 16
        ridx[pl.ds(c * 16, 16)] = rbase + lax.shift_right_logical(ii, 3)
        return carry

    lax.fori_loop(0, _NV // 16, _mk_ridx, 0, unroll=4)

    def _fill_zero(g, carry):
        aggv[pl.ds(g * 16, 16)] = jnp.zeros((16,), jnp.float32)
        return carry

    lax.fori_loop(0, _NPT // 16, _fill_zero, 0, unroll=4)
    pltpu.sync_copy(aggv, agg_sh.at[pl.ds(rbase, _NPT)])

    for cp in cps:
        cp.wait()

    def _mk_ratio(c, carry):
        sl = pl.ds(c * 16, 16)
        rat[sl] = (sa[sl] + sb[sl]) / jnp.maximum(ca[sl] + cb[sl], 1.0)
        return carry

    lax.fori_loop(0, _NV // 16, _mk_ratio, 0, unroll=4)

    pltpu.sync_copy(rat, agg_sh.at[ridx], add=True)
    pltpu.sync_copy(agg_sh.at[pl.ds(rbase, _NPT)], aggv)
    gat.wait()

    def _mk_score(g, carry):
        sl = pl.ds(g * 16, 16)
        val = aggv[sl] + xrv[sl] + bv[...]
        e = jnp.exp(-2.0 * jnp.abs(val))
        th = (1.0 - e) / (1.0 + e)
        aggv[sl] = jnp.where(val < 0.0, -th, th)
        return carry

    lax.fori_loop(0, _NPT // 16, _mk_score, 0, unroll=4)
    cpo = pltpu.async_copy(aggv, sc_out.at[pl.ds(n0, _NPT)], sem2)
    pltpu.sync_copy(aggv, sp_out.at[pidx])
    cpo.wait()


def _scale_body(x_ref, s_ref, xo_ref):
    xo_ref[...] = x_ref[...] * s_ref[:, :1]


def _mm_body(x_ref, w_ref, r_ref, z_ref):
    xb = x_ref[...]
    yb = lax.dot_general(xb, w_ref[...], (((1,), (1,)), ((), ())),
                         preferred_element_type=jnp.float32)
    xr = jnp.dot(xb, r_ref[...], preferred_element_type=jnp.float32)
    z_ref[...] = jnp.concatenate(
        [yb, xr, jnp.zeros((_BN, _D - _R - 1), jnp.float32)], axis=1)


def kernel(x, edge_index, edge_attr, weight, root, bias):
    src = edge_index[0].astype(jnp.int32)
    dst = edge_index[1].astype(jnp.int32)
    rel = edge_attr.astype(jnp.int32)

    c_part, gi, si = _sc_counts(src, dst, rel)

    w2 = weight.reshape(_R, _D)
    z = pl.pallas_call(
        _mm_body,
        grid=(_GRID,),
        in_specs=[
            pl.BlockSpec((_BN, _D), lambda i: (i, 0)),
            pl.BlockSpec((_R, _D), lambda i: (0, 0)),
            pl.BlockSpec((_D, 1), lambda i: (0, 0)),
        ],
        out_specs=pl.BlockSpec((_BN, _D), lambda i: (i, 0)),
        out_shape=jax.ShapeDtypeStruct((_N, _D), jnp.float32),
    )(x, w2, root)

    z_flat = z.reshape(_N * _D)

    s_part = _sc_sums(gi, si, z_flat)

    bias16 = jnp.broadcast_to(bias, (16,))
    score, spad = _sc_score(s_part, c_part, z_flat, bias16)

    x_out = pl.pallas_call(
        _scale_body,
        grid=(_GRID,),
        in_specs=[
            pl.BlockSpec((_BN, _D), lambda i: (i, 0)),
            pl.BlockSpec((_BN, _D), lambda i: (i, 0)),
        ],
        out_specs=pl.BlockSpec((_BN, _D), lambda i: (i, 0)),
        out_shape=jax.ShapeDtypeStruct((_N, _D), jnp.float32),
    )(x, spad.reshape(_N, _D))

    return (x_out, score)


# speedup vs baseline: 1.0368x; 1.0368x over previous
import functools

import jax
import jax.numpy as jnp
from jax import lax
from jax.experimental import pallas as pl
from jax.experimental.pallas import tpu as pltpu
from jax.experimental.pallas import tpu_sc as plsc

_N = 10000
_E = 320000
_D = 128
_R = 8
_NR = _N * _R

_NC = 2
_NS = 16
_NW = _NC * _NS
_EPW = _E // _NW
_CH16 = _EPW // 16
_SEG = _NR // _NS
_ZB = ((_SEG + 15) // 16) * 16
_NCHUNK = 5
_CHW = _EPW // _NCHUNK

_BN = 2000
_GRID = _N // _BN

_mesh = plsc.VectorSubcoreMesh(core_axis_name="c", subcore_axis_name="s")


@functools.partial(
    pl.kernel,
    out_type=[
        jax.ShapeDtypeStruct((_NC * _NR,), jnp.float32),
        jax.ShapeDtypeStruct((_E,), jnp.int32),
        jax.ShapeDtypeStruct((_E,), jnp.int32),
    ],
    mesh=_mesh,
    scratch_types=[
        pltpu.VMEM((_EPW,), jnp.int32),
        pltpu.VMEM((_EPW,), jnp.int32),
        pltpu.VMEM((_EPW,), jnp.int32),
        pltpu.VMEM((_EPW,), jnp.float32),
        pltpu.VMEM((_ZB,), jnp.float32),
        pltpu.VMEM_SHARED((_NR,), jnp.float32),
        pltpu.SemaphoreType.DMA,
        pltpu.SemaphoreType.DMA,
    ],
)
def _sc_counts(src_hbm, dst_hbm, rel_hbm,
               c_out, gi_out, si_out,
               gidx, sidx, rel_v, ones_v, zbuf, c_sh, sem, sem2):
    cid = lax.axis_index("c")
    sid = lax.axis_index("s")
    wid = sid * _NC + cid

    base = wid * _EPW
    cp_src = pltpu.async_copy(src_hbm.at[pl.ds(base, _EPW)], gidx, sem2)
    cp_dst = pltpu.async_copy(dst_hbm.at[pl.ds(base, _EPW)], sidx, sem2)
    cp_rel = pltpu.async_copy(rel_hbm.at[pl.ds(base, _EPW)], rel_v, sem2)

    def _fill_zero(i, carry):
        zbuf[pl.ds(i * 16, 16)] = jnp.zeros((16,), jnp.float32)
        return carry

    lax.fori_loop(0, _ZB // 16, _fill_zero, 0, unroll=4)

    def _fill_one(i, carry):
        ones_v[pl.ds(i * 16, 16)] = jnp.full((16,), 1.0, jnp.float32)
        return carry

    lax.fori_loop(0, _CH16, _fill_one, 0, unroll=4)

    off = sid * _SEG
    pltpu.sync_copy(zbuf.at[pl.ds(0, _SEG)], c_sh.at[pl.ds(off, _SEG)])

    cp_src.wait()
    cp_dst.wait()
    cp_rel.wait()

    def _mk_sidx(i, carry):
        sl = pl.ds(i * 16, 16)
        sidx[sl] = sidx[sl] * _R + rel_v[sl]
        return carry

    lax.fori_loop(0, _CH16, _mk_sidx, 0, unroll=4)
    cp_si = pltpu.async_copy(sidx, si_out.at[pl.ds(base, _EPW)], sem2)

    plsc.subcore_barrier()

    sct = pltpu.async_copy(ones_v, c_sh.at[sidx], sem, add=True)

    def _mk_gidx(i, carry):
        sl = pl.ds(i * 16, 16)
        gidx[sl] = gidx[sl] * _D + rel_v[sl]
        return carry

    lax.fori_loop(0, _CH16, _mk_gidx, 0, unroll=4)
    cp_gi = pltpu.async_copy(gidx, gi_out.at[pl.ds(base, _EPW)], sem2)

    sct.wait()
    cp_gi.wait()
    cp_si.wait()
    plsc.subcore_barrier()

    oo = cid * _NR + off
    pltpu.sync_copy(c_sh.at[pl.ds(off, _SEG)], ones_v.at[pl.ds(0, _SEG)])
    pltpu.sync_copy(ones_v.at[pl.ds(0, _SEG)], c_out.at[pl.ds(oo, _SEG)])


@functools.partial(
    pl.kernel,
    out_type=jax.ShapeDtypeStruct((_NC * _NR,), jnp.float32),
    mesh=_mesh,
    scratch_types=[
        pltpu.VMEM((_NCHUNK, 1, _CHW), jnp.int32),
        pltpu.VMEM((_NCHUNK, 1, _CHW), jnp.int32),
        pltpu.VMEM((_NCHUNK, 1, _CHW), jnp.float32),
        pltpu.VMEM((_ZB,), jnp.float32),
        pltpu.VMEM_SHARED((_NR,), jnp.float32),
        pltpu.SemaphoreType.DMA,
        pltpu.SemaphoreType.DMA,
    ],
)
def _sc_sums(gi_hbm, si_hbm, z_hbm, s_out,
             gidx, sidx, vals, zbuf, s_sh, sem, sem2):
    cid = lax.axis_index("c")
    sid = lax.axis_index("s")
    wid = sid * _NC + cid

    base = wid * _EPW
    cps = []
    for k in range(_NCHUNK):
        cps.append(pltpu.async_copy(
            gi_hbm.at[pl.ds(base + k * _CHW, _CHW)], gidx.at[k, 0], sem2))
        cps.append(pltpu.async_copy(
            si_hbm.at[pl.ds(base + k * _CHW, _CHW)], sidx.at[k, 0], sem2))

    def _fill_zero(i, carry):
        zbuf[pl.ds(i * 16, 16)] = jnp.zeros((16,), jnp.float32)
        return carry

    lax.fori_loop(0, _ZB // 16, _fill_zero, 0, unroll=4)

    off = sid * _SEG
    pltpu.sync_copy(zbuf.at[pl.ds(0, _SEG)], s_sh.at[pl.ds(off, _SEG)])

    for cp in cps:
        cp.wait()

    gats = [pltpu.async_copy(z_hbm.at[gidx.at[k, 0]], vals.at[k, 0], sem)
            for k in range(2)]

    plsc.subcore_barrier()

    scts = []
    for k in range(_NCHUNK):
        if k + 2 < _NCHUNK:
            gats.append(pltpu.async_copy(
                z_hbm.at[gidx.at[k + 2, 0]], vals.at[k + 2, 0], sem))
        gats[k].wait()
        scts.append(pltpu.async_copy(
            vals.at[k, 0], s_sh.at[sidx.at[k, 0]], sem2, add=True))
    for s in scts:
        s.wait()

    plsc.subcore_barrier()

    oo = cid * _NR + off
    pltpu.sync_copy(s_sh.at[pl.ds(off, _SEG)], zbuf.at[pl.ds(0, _SEG)])
    pltpu.sync_copy(zbuf.at[pl.ds(0, _SEG)], s_out.at[pl.ds(oo, _SEG)])


_NPT = 320
_NLAST = _N - _NPT
_NV = _NPT * _R


@functools.partial(
    pl.kernel,
    out_type=jax.ShapeDtypeStruct((_N,), jnp.float32),
    mesh=_mesh,
    scratch_types=[
        pltpu.VMEM((_NV,), jnp.float32),
        pltpu.VMEM((_NV,), jnp.float32),
        pltpu.VMEM((_NV,), jnp.float32),
        pltpu.VMEM((_NV,), jnp.float32),
        pltpu.VMEM((_NV,), jnp.float32),
        pltpu.VMEM((_NV,), jnp.int32),
        pltpu.VMEM((_NPT,), jnp.float32),
        pltpu.VMEM((_NPT,), jnp.int32),
        pltpu.VMEM((_NPT,), jnp.int32),
        pltpu.VMEM((_NPT,), jnp.float32),
        pltpu.VMEM((16,), jnp.float32),
        pltpu.VMEM_SHARED((_NS * _NPT,), jnp.float32),
        pltpu.SemaphoreType.DMA,
        pltpu.SemaphoreType.DMA,
    ],
)
def _sc_score(s_hbm, c_hbm, z_hbm, b_hbm, sc_out,
              sa, sb, ca, cb, rat, ridx, aggv, xidx, pidx, xrv, bv, agg_sh,
              sem, sem2):
    cid = lax.axis_index("c")
    sid = lax.axis_index("s")
    wid = sid * _NC + cid
    n0 = jnp.minimum(wid * _NPT, _NLAST)

    cps = [
        pltpu.async_copy(s_hbm.at[pl.ds(n0 * _R, _NV)], sa, sem2),
        pltpu.async_copy(s_hbm.at[pl.ds(_NR + n0 * _R, _NV)], sb, sem2),
        pltpu.async_copy(c_hbm.at[pl.ds(n0 * _R, _NV)], ca, sem2),
        pltpu.async_copy(c_hbm.at[pl.ds(_NR + n0 * _R, _NV)], cb, sem2),
        pltpu.async_copy(b_hbm, bv, sem2),
    ]

    iota16 = lax.iota(jnp.int32, 16)

    def _mk_xidx(g, carry):
        t = (iota16 + g * 16 + n0) * _D
        xidx[pl.ds(g * 16, 16)] = t + _R
        pidx[pl.ds(g * 16, 16)] = t
        return carry

    lax.fori_loop(0, _NPT // 16, _mk_xidx, 0, unroll=4)
    gat = pltpu.async_copy(z_hbm.at[xidx], xrv, sem)

    rbase = sid * _NPT

    def _mk_ridx(c, carry):
        ii = iota16 + c * 16
        ridx[pl.ds(c * 16, 16)] = rbase + lax.shift_right_logical(ii, 3)
        return carry

    lax.fori_loop(0, _NV // 16, _mk_ridx, 0, unroll=4)

    def _fill_zero(g, carry):
        aggv[pl.ds(g * 16, 16)] = jnp.zeros((16,), jnp.float32)
        return carry

    lax.fori_loop(0, _NPT // 16, _fill_zero, 0, unroll=4)
    pltpu.sync_copy(aggv, agg_sh.at[pl.ds(rbase, _NPT)])

    for cp in cps:
        cp.wait()

    def _mk_ratio(c, carry):
        sl = pl.ds(c * 16, 16)
        rat[sl] = (sa[sl] + sb[sl]) / jnp.maximum(ca[sl] + cb[sl], 1.0)
        return carry

    lax.fori_loop(0, _NV // 16, _mk_ratio, 0, unroll=4)

    pltpu.sync_copy(rat, agg_sh.at[ridx], add=True)
    pltpu.sync_copy(agg_sh.at[pl.ds(rbase, _NPT)], aggv)
    gat.wait()

    def _mk_score(g, carry):
        sl = pl.ds(g * 16, 16)
        val = aggv[sl] + xrv[sl] + bv[...]
        e = jnp.exp(-2.0 * jnp.abs(val))
        th = (1.0 - e) / (1.0 + e)
        aggv[sl] = jnp.where(val < 0.0, -th, th)
        return carry

    lax.fori_loop(0, _NPT // 16, _mk_score, 0, unroll=4)
    pltpu.sync_copy(aggv, sc_out.at[pl.ds(n0, _NPT)])


def _scale_body(x_ref, s_ref, xo_ref):
    xo_ref[...] = x_ref[...] * s_ref[...]


def _mm_body(x_ref, w_ref, r_ref, z_ref):
    xb = x_ref[...]
    yb = lax.dot_general(xb, w_ref[...], (((1,), (1,)), ((), ())),
                         preferred_element_type=jnp.float32)
    xr = jnp.dot(xb, r_ref[...], preferred_element_type=jnp.float32)
    z_ref[...] = jnp.concatenate(
        [yb, xr, jnp.zeros((_BN, _D - _R - 1), jnp.float32)], axis=1)


def kernel(x, edge_index, edge_attr, weight, root, bias):
    src = edge_index[0].astype(jnp.int32)
    dst = edge_index[1].astype(jnp.int32)
    rel = edge_attr.astype(jnp.int32)

    c_part, gi, si = _sc_counts(src, dst, rel)

    w2 = weight.reshape(_R, _D)
    z = pl.pallas_call(
        _mm_body,
        grid=(_GRID,),
        in_specs=[
            pl.BlockSpec((_BN, _D), lambda i: (i, 0)),
            pl.BlockSpec((_R, _D), lambda i: (0, 0)),
            pl.BlockSpec((_D, 1), lambda i: (0, 0)),
        ],
        out_specs=pl.BlockSpec((_BN, _D), lambda i: (i, 0)),
        out_shape=jax.ShapeDtypeStruct((_N, _D), jnp.float32),
    )(x, w2, root)

    z_flat = z.reshape(_N * _D)

    s_part = _sc_sums(gi, si, z_flat)

    bias16 = jnp.broadcast_to(bias, (16,))
    score = _sc_score(s_part, c_part, z_flat, bias16)

    x_out = pl.pallas_call(
        _scale_body,
        grid=(_GRID,),
        in_specs=[
            pl.BlockSpec((_BN, _D), lambda i: (i, 0)),
            pl.BlockSpec((_BN, 1), lambda i: (i, 0)),
        ],
        out_specs=pl.BlockSpec((_BN, _D), lambda i: (i, 0)),
        out_shape=jax.ShapeDtypeStruct((_N, _D), jnp.float32),
    )(x, score.reshape(_N, 1))

    return (x_out, score)
